# Initial kernel scaffold; baseline (speedup 1.0000x reference)
import jax
import jax.numpy as jnp
from jax.experimental import pallas as pl


def kernel(cls_p3, cls_p4, cls_p5, cls_p6, cls_p7, cen_p3, cen_p4, cen_p5, cen_p6, cen_p7, reg_p3, reg_p4, reg_p5, reg_p6, reg_p7):
    raise NotImplementedError("write your pallas kernel here")


# speedup vs baseline: 20.3390x; 20.3390x over previous
import functools

import jax
import jax.numpy as jnp
from jax.experimental import pallas as pl
from jax.experimental.pallas import tpu as pltpu

_STRIDES = [8, 16, 32, 64, 128]
_SIZES = [(64, 64), (32, 32), (16, 16), (8, 8), (4, 4)]
_SCORE_THR = 0.05
_NMS_THR = 0.6
_MAX_BOX = 1000
_NP = 1024


def _reshape_cat(levels):
    outs = []
    for t in levels:
        B, C, H, W = t.shape
        outs.append(jnp.transpose(t, (0, 2, 3, 1)).reshape(B, H * W, C))
    return jnp.concatenate(outs, axis=1)


def _coords():
    cs = []
    for (h, w), s in zip(_SIZES, _STRIDES):
        xs = (jnp.arange(w, dtype=jnp.float32) + 0.5) * s
        ys = (jnp.arange(h, dtype=jnp.float32) + 0.5) * s
        yy, xx = jnp.meshgrid(ys, xs, indexing='ij')
        cs.append(jnp.stack([xx.reshape(-1), yy.reshape(-1)], axis=-1))
    return jnp.concatenate(cs, axis=0)


def _nms_body(vals_ref, bc_ref, br_ref, keep_ref, m_ref):
    x1c = bc_ref[0, :, 0:1]
    y1c = bc_ref[0, :, 1:2]
    x2c = bc_ref[0, :, 2:3]
    y2c = bc_ref[0, :, 3:4]
    x1r = br_ref[0, 0:1, :]
    y1r = br_ref[0, 1:2, :]
    x2r = br_ref[0, 2:3, :]
    y2r = br_ref[0, 3:4, :]
    wx = jnp.maximum(jnp.minimum(x2c, x2r) - jnp.maximum(x1c, x1r), 0.0)
    wy = jnp.maximum(jnp.minimum(y2c, y2r) - jnp.maximum(y1c, y1r), 0.0)
    inter = wx * wy
    areac = jnp.maximum(x2c - x1c, 0.0) * jnp.maximum(y2c - y1c, 0.0)
    arear = jnp.maximum(x2r - x1r, 0.0) * jnp.maximum(y2r - y1r, 0.0)
    union = areac + arear - inter
    iou = inter / (union + 1e-9)
    row_j = jax.lax.broadcasted_iota(jnp.int32, (_NP, _NP), 0)
    col_i = jax.lax.broadcasted_iota(jnp.int32, (_NP, _NP), 1)
    m_ref[...] = jnp.where((iou > _NMS_THR) & (col_i < row_j), 1.0, 0.0)

    validf = (vals_ref[0] >= _SCORE_THR).astype(jnp.float32)

    def cond(c):
        return ~c[1]

    def body(c):
        keep, _ = c
        sup = jax.lax.dot_general(
            keep, m_ref[...], (((1,), (1,)), ((), ())),
            preferred_element_type=jnp.float32)
        new = validf * (sup == 0.0).astype(jnp.float32)
        return new, jnp.all(new == keep)

    keep, _ = jax.lax.while_loop(cond, body, (validf, False))
    keep_ref[0] = keep


def _nms_pallas(vals, boxes_off):
    B = vals.shape[0]
    br = jnp.transpose(boxes_off, (0, 2, 1))
    vals3 = vals.reshape(B, 1, _NP)
    grid = (B,)
    out = pl.pallas_call(
        _nms_body,
        grid=grid,
        in_specs=[
            pl.BlockSpec((1, 1, _NP), lambda b: (b, 0, 0)),
            pl.BlockSpec((1, _NP, 4), lambda b: (b, 0, 0)),
            pl.BlockSpec((1, 4, _NP), lambda b: (b, 0, 0)),
        ],
        out_specs=pl.BlockSpec((1, 1, _NP), lambda b: (b, 0, 0)),
        out_shape=jax.ShapeDtypeStruct((B, 1, _NP), jnp.float32),
        scratch_shapes=[pltpu.VMEM((_NP, _NP), jnp.float32)],
    )(vals3, boxes_off, br)
    return out.reshape(B, _NP)


def kernel(cls_p3, cls_p4, cls_p5, cls_p6, cls_p7,
           cen_p3, cen_p4, cen_p5, cen_p6, cen_p7,
           reg_p3, reg_p4, reg_p5, reg_p6, reg_p7):
    cls_logits = _reshape_cat([cls_p3, cls_p4, cls_p5, cls_p6, cls_p7])
    cen_logits = _reshape_cat([cen_p3, cen_p4, cen_p5, cen_p6, cen_p7])
    reg_preds = _reshape_cat([reg_p3, reg_p4, reg_p5, reg_p6, reg_p7])
    coords = _coords()
    cls_preds = jax.nn.sigmoid(cls_logits)
    cen_preds = jax.nn.sigmoid(cen_logits)
    cls_score = jnp.max(cls_preds, axis=-1)
    cls_classes = jnp.argmax(cls_preds, axis=-1) + 1
    cls_score = jnp.sqrt(cls_score * jnp.squeeze(cen_preds, axis=-1))
    x1y1 = coords[None, :, :] - reg_preds[..., :2]
    x2y2 = coords[None, :, :] + reg_preds[..., 2:]
    boxes = jnp.concatenate([x1y1, x2y2], axis=-1)
    max_num = min(_MAX_BOX, cls_score.shape[1])
    topv, topi = jax.lax.top_k(cls_score, max_num)
    classes_k = jnp.take_along_axis(cls_classes, topi, axis=1)
    boxes_k = jnp.take_along_axis(boxes, topi[..., None], axis=1)

    max_coord = jnp.max(boxes_k, axis=(1, 2), keepdims=True)
    off = classes_k.astype(jnp.float32)[..., None] * (max_coord + 1.0)
    boxes_off = boxes_k + off

    vals_p = jnp.pad(topv, ((0, 0), (0, _NP - max_num)), constant_values=-1.0)
    boxes_p = jnp.pad(boxes_off, ((0, 0), (0, _NP - max_num), (0, 0)))

    keep = _nms_pallas(vals_p, boxes_p)[:, :max_num]
    keepb = keep > 0.0
    return (topv * keep, classes_k * keepb, boxes_k * keep[..., None])
